# baseline (speedup 1.0000x reference)
import functools

import jax
import jax.numpy as jnp
from jax.experimental import pallas as pl
from jax.experimental.pallas import tpu as pltpu

_R = 5
_D = 16
_B = 16384
_BLK = 2048
_G = _B // _BLK


def _tc_body(zu_ref, zi_ref, qc_ref, xui_ref, pui_ref):
    zu = zu_ref[...]
    y = jnp.dot(zi_ref[...], qc_ref[...], preferred_element_type=jnp.float32)
    ps = [jnp.sum(zu * y[:, r * _D:(r + 1) * _D], axis=1) for r in range(_R)]
    m = jnp.maximum(jnp.maximum(jnp.maximum(ps[0], ps[1]),
                                jnp.maximum(ps[2], ps[3])), ps[4])
    es = [jnp.exp(p - m) for p in ps]
    s = (es[0] + es[1]) + (es[2] + es[3]) + es[4]
    num = (es[1] + 2.0 * es[2]) + (3.0 * es[3] + 4.0 * es[4])
    xui_ref[...] = num / s
    pui_ref[...] = jnp.concatenate([p[:, None] for p in ps], axis=1)


@jax.jit
def kernel(zu, zi, Q):
    qc = Q.transpose(1, 0, 2).reshape(_D, _R * _D)
    grid_spec = pl.GridSpec(
        grid=(_G,),
        in_specs=[
            pl.BlockSpec((_BLK, _D), lambda i: (i, 0)),
            pl.BlockSpec((_BLK, _D), lambda i: (i, 0)),
            pl.BlockSpec((_D, _R * _D), lambda i: (0, 0)),
        ],
        out_specs=[
            pl.BlockSpec((_BLK,), lambda i: (i,)),
            pl.BlockSpec((_BLK, _R), lambda i: (i, 0)),
        ],
    )
    xui, pui = pl.pallas_call(
        _tc_body,
        grid_spec=grid_spec,
        out_shape=[
            jax.ShapeDtypeStruct((_B,), jnp.float32),
            jax.ShapeDtypeStruct((_B, _R), jnp.float32),
        ],
        compiler_params=pltpu.CompilerParams(
            dimension_semantics=("arbitrary",),
        ),
    )(zu, zi, qc)
    return (xui, pui)


# speedup vs baseline: 1.5583x; 1.5583x over previous
import functools

import jax
import jax.numpy as jnp
import numpy as np
from jax.experimental import pallas as pl
from jax.experimental.pallas import tpu as pltpu

_R = 5
_D = 16
_B = 16384
_BLK = 4096
_G = _B // _BLK

_K_TILE = np.tile(np.eye(_D, dtype=np.float32), (1, _R))
_S_IND = np.repeat(np.eye(_R, dtype=np.float32), _D, axis=0)


def _mm(a, b):
    return jnp.dot(a, b, preferred_element_type=jnp.float32)


def _tc_body(zu_ref, zi_ref, qc_ref, k_ref, s_ref, xui_ref, pui_ref):
    y = _mm(zi_ref[...], qc_ref[...])
    u = _mm(zu_ref[...], k_ref[...])
    p5 = _mm(y * u, s_ref[...])
    m = jnp.max(p5, axis=1, keepdims=True)
    es = jnp.exp(p5 - m)
    w_exp = jax.lax.broadcasted_iota(jnp.int32, (_R, 1), 0).astype(jnp.float32)
    s = _mm(es, jnp.ones((_R, 1), jnp.float32))
    num = _mm(es, w_exp)
    xui_ref[...] = num / s
    pui_ref[...] = p5


@jax.jit
def kernel(zu, zi, Q):
    qc = Q.transpose(1, 0, 2).reshape(_D, _R * _D)
    grid_spec = pl.GridSpec(
        grid=(_G,),
        in_specs=[
            pl.BlockSpec((_BLK, _D), lambda i: (i, 0)),
            pl.BlockSpec((_BLK, _D), lambda i: (i, 0)),
            pl.BlockSpec((_D, _R * _D), lambda i: (0, 0)),
            pl.BlockSpec((_D, _R * _D), lambda i: (0, 0)),
            pl.BlockSpec((_R * _D, _R), lambda i: (0, 0)),
        ],
        out_specs=[
            pl.BlockSpec((_BLK, 1), lambda i: (i, 0)),
            pl.BlockSpec((_BLK, _R), lambda i: (i, 0)),
        ],
    )
    xui, pui = pl.pallas_call(
        _tc_body,
        grid_spec=grid_spec,
        out_shape=[
            jax.ShapeDtypeStruct((_B, 1), jnp.float32),
            jax.ShapeDtypeStruct((_B, _R), jnp.float32),
        ],
        compiler_params=pltpu.CompilerParams(
            dimension_semantics=("arbitrary",),
        ),
    )(zu, zi, qc, jnp.asarray(_K_TILE), jnp.asarray(_S_IND))
    return (xui.reshape(_B), pui)
